# baseline (speedup 1.0000x reference)
import functools

import jax
import jax.numpy as jnp
from jax import lax
from jax.experimental import pallas as pl
from jax.experimental.pallas import tpu as pltpu
from jax.experimental.pallas import tpu_sc as plsc

_V = 1000000
_D = 64
_B = 4096
_H = 200
_FULL_BLOCKS = (_V // 128)
_TAIL = _V - _FULL_BLOCKS * 128
_PAIR_ROWS = _V // 2


def _mask_body(t_ref, m_ref):
    m_ref[...] = t_ref[...] == 0


def _pairize_kernel(table_t, tail_lin):
    mesh = plsc.VectorSubcoreMesh(core_axis_name="c", subcore_axis_name="s")
    info = plsc.get_sparse_core_info()
    NC, NS = info.num_cores, info.num_subcores
    NW = NC * NS
    n_iter = (_FULL_BLOCKS + NW - 1) // NW

    @functools.partial(
        pl.kernel,
        mesh=mesh,
        compiler_params=pltpu.CompilerParams(use_tc_tiling_on_sc=True, needs_layout_passes=False),
        out_type=jax.ShapeDtypeStruct((_PAIR_ROWS, 128), jnp.float32),
        scratch_types=[
            pltpu.VMEM((_D, 128), jnp.float32),
            pltpu.VMEM((_D, 128), jnp.float32),
            pltpu.VMEM((_TAIL * _D,), jnp.float32),
            pltpu.VMEM((_TAIL // 2, 128), jnp.float32),
        ],
    )
    def k(tab_hbm, tail_hbm, pair_hbm, blk, t2, tl1, tl2):
        wid = lax.axis_index("s") * NC + lax.axis_index("c")
        iota = lax.iota(jnp.int32, 16)

        def step(i, carry):
            blk_id = i * NW + wid

            @pl.when(blk_id < _FULL_BLOCKS)
            def _():
                pltpu.sync_copy(tab_hbm.at[:, pl.ds(blk_id * 128, 128)], blk)

                def trans(q, c):
                    for k0 in range(8):
                        rows = iota + (16 * (k0 % 4))
                        cols = jnp.full((16,), 2 * q + (1 if k0 >= 4 else 0),
                                        jnp.int32)
                        v = plsc.load_gather(blk, [rows, cols])
                        t2[q, pl.ds(k0 * 16, 16)] = v
                    return c

                lax.fori_loop(0, _D, trans, 0)
                pltpu.sync_copy(t2, pair_hbm.at[pl.ds(blk_id * 64, 64), :])

            return carry

        lax.fori_loop(0, n_iter, step, 0)

        @pl.when(wid == 0)
        def _():
            pltpu.sync_copy(tail_hbm, tl1)

            def tcopy(r, c):
                for j0 in range(8):
                    tl2[r, pl.ds(j0 * 16, 16)] = tl1[pl.ds(r * 128 + j0 * 16, 16)]
                return c

            lax.fori_loop(0, _TAIL // 2, tcopy, 0)
            pltpu.sync_copy(
                tl2, pair_hbm.at[pl.ds(_FULL_BLOCKS * 64, _TAIL // 2), :])

    return k(table_t, tail_lin)


def _gather_kernel(term_t, pair):
    mesh = plsc.VectorSubcoreMesh(core_axis_name="c", subcore_axis_name="s")
    info = plsc.get_sparse_core_info()
    NC, NS = info.num_cores, info.num_subcores
    NW = NC * NS
    n_ht = _H // 8
    n_bb = _B // 128
    per_w = (n_ht * n_bb) // NW

    @functools.partial(
        pl.kernel,
        mesh=mesh,
        compiler_params=pltpu.CompilerParams(use_tc_tiling_on_sc=True, needs_layout_passes=False),
        out_type=jax.ShapeDtypeStruct((_H, _D, _B), jnp.float32),
        scratch_types=[
            pltpu.VMEM((8, 128), jnp.int32),
            pltpu.VMEM((128,), jnp.int32),
            pltpu.VMEM((128,), jnp.int32),
            pltpu.VMEM((128, 128), jnp.float32),
            pltpu.VMEM((_D, 128), jnp.float32),
            pltpu.SemaphoreType.DMA,
        ],
    )
    def k(term_hbm, pair_hbm, out_hbm, itile, pidx, off, G, t, gsem):
        wid = lax.axis_index("s") * NC + lax.axis_index("c")
        iota = lax.iota(jnp.int32, 16)

        def step(i, carry):
            e = wid * per_w + i
            ht = e // n_bb
            bb = e - ht * n_bb
            pltpu.sync_copy(
                term_hbm.at[pl.ds(ht * 8, 8), pl.ds(bb * 128, 128)], itile)

            for h_sub in range(8):
                def prep(j0, c):
                    r = itile[h_sub, pl.ds(j0 * 16, 16)]
                    pidx[pl.ds(j0 * 16, 16)] = lax.shift_right_logical(r, 1)
                    off[pl.ds(j0 * 16, 16)] = (r & 1) * 64
                    return c

                lax.fori_loop(0, 8, prep, 0)
                pltpu.async_copy(pair_hbm.at[pidx], G, gsem).wait()

                def trans(d, c):
                    for j0 in range(8):
                        rows = iota + (16 * j0)
                        cols = off[pl.ds(j0 * 16, 16)] + d
                        t[d, pl.ds(j0 * 16, 16)] = plsc.load_gather(
                            G, [rows, cols])
                    return c

                lax.fori_loop(0, _D, trans, 0)
                pltpu.sync_copy(
                    t, out_hbm.at[ht * 8 + h_sub, :, pl.ds(bb * 128, 128)])
            return carry

        lax.fori_loop(0, per_w, step, 0)

    return k(term_t, pair)


def kernel(term, table):
    tail_lin = jnp.reshape(
        lax.slice(table, (_FULL_BLOCKS * 128, 0), (_V, _D)), (_TAIL * _D,))
    pair = _pairize_kernel(table.T, tail_lin)
    emb_t = _gather_kernel(term.T, pair)
    emb = jnp.transpose(emb_t, (2, 0, 1))

    mask_t = pl.pallas_call(
        _mask_body,
        out_shape=jax.ShapeDtypeStruct((_H, _B), jnp.bool_),
    )(term.T)
    return emb, mask_t.T


# speedup vs baseline: 1.7693x; 1.7693x over previous
import functools

import jax
import jax.numpy as jnp
from jax import lax
from jax.experimental import pallas as pl
from jax.experimental.pallas import tpu as pltpu
from jax.experimental.pallas import tpu_sc as plsc

_V = 1000000
_D = 64
_B = 4096
_H = 200
_PAIR_ROWS = _V // 2
_STRIDE = 129


def _mask_body(t_ref, m_ref):
    m_ref[...] = t_ref[...] == 0


def _gather_kernel(term_t, pair):
    mesh = plsc.VectorSubcoreMesh(core_axis_name="c", subcore_axis_name="s")
    info = plsc.get_sparse_core_info()
    NC, NS = info.num_cores, info.num_subcores
    NW = NC * NS
    n_ht = _H // 8
    n_bb = _B // 128
    per_w = (n_ht * n_bb) // NW

    @functools.partial(
        pl.kernel,
        mesh=mesh,
        compiler_params=pltpu.CompilerParams(
            use_tc_tiling_on_sc=True, needs_layout_passes=False),
        out_type=jax.ShapeDtypeStruct((_H, _D, _B), jnp.float32),
        scratch_types=[
            pltpu.VMEM((8, 128), jnp.int32),
            pltpu.VMEM((2, 128), jnp.int32),
            pltpu.VMEM((128,), jnp.int32),
            pltpu.VMEM((2, 128, 128), jnp.float32),
            pltpu.VMEM((128 * _STRIDE,), jnp.float32),
            pltpu.VMEM((2, _D, 128), jnp.float32),
            pltpu.SemaphoreType.DMA,
            pltpu.SemaphoreType.DMA,
        ],
    )
    def k(term_hbm, pair_hbm, out_hbm, itile, pidx, basev, G, G1, t,
          gsem, osem):
        wid = lax.axis_index("s") * NC + lax.axis_index("c")
        iota = lax.iota(jnp.int32, 16)

        def load_itile(e):
            ht = e // n_bb
            bb = e - ht * n_bb
            pltpu.sync_copy(
                term_hbm.at[pl.ds(ht * 8, 8), pl.ds(bb * 128, 128)], itile)

        def fire(h_sub, slot):
            def prep(j0, c):
                r = itile[h_sub, pl.ds(j0 * 16, 16)]
                pidx[slot, pl.ds(j0 * 16, 16)] = lax.shift_right_logical(r, 1)
                return c
            lax.fori_loop(0, 8, prep, 0)
            pltpu.async_copy(pair_hbm.at[pidx.at[slot]], G.at[slot], gsem)

        def process(h_sub, slot, ht, bb):
            pltpu.make_async_copy(
                pair_hbm.at[pidx.at[slot]], G.at[slot], gsem).wait()
            def mkbase(j0, c):
                r = itile[h_sub, pl.ds(j0 * 16, 16)]
                basev[pl.ds(j0 * 16, 16)] = (
                    (iota + j0 * 16) * _STRIDE + (r & 1) * 64)
                return c
            lax.fori_loop(0, 8, mkbase, 0)

            def stage(j, c):
                for c0 in range(8):
                    G1[pl.ds(j * _STRIDE + c0 * 16, 16)] = (
                        G[slot, j, pl.ds(c0 * 16, 16)])
                return c
            lax.fori_loop(0, 128, stage, 0)

            tb = t.at[slot]

            def trans(d, c):
                for j0 in range(8):
                    idxv = basev[pl.ds(j0 * 16, 16)] + d
                    tb[d, pl.ds(j0 * 16, 16)] = plsc.load_gather(G1, [idxv])
                return c
            lax.fori_loop(0, _D, trans, 0)
            pltpu.async_copy(
                tb, out_hbm.at[ht * 8 + h_sub, :, pl.ds(bb * 128, 128)], osem)

        def step(i, carry):
            e = wid * per_w + i
            ht = e // n_bb
            bb = e - ht * n_bb
            load_itile(e)
            fire(0, 0)
            for h_sub in range(8):
                slot = h_sub % 2
                if h_sub + 1 < 8:
                    fire(h_sub + 1, 1 - slot)
                if h_sub >= 2:
                    h_prev = h_sub - 2
                    pltpu.make_async_copy(
                        t.at[h_prev % 2],
                        out_hbm.at[ht * 8 + h_prev, :, pl.ds(bb * 128, 128)],
                        osem).wait()
                process(h_sub, slot, ht, bb)
            for h_prev in (6, 7):
                pltpu.make_async_copy(
                    t.at[h_prev % 2],
                    out_hbm.at[ht * 8 + h_prev, :, pl.ds(bb * 128, 128)],
                    osem).wait()
            return carry

        lax.fori_loop(0, per_w, step, 0)

    return k(term_t, pair)


def kernel(term, table):
    pair = jnp.reshape(table, (_PAIR_ROWS, 128))
    emb_t = _gather_kernel(term.T, pair)
    emb = jnp.transpose(emb_t, (2, 0, 1))

    mask_t = pl.pallas_call(
        _mask_body,
        out_shape=jax.ShapeDtypeStruct((_H, _B), jnp.bool_),
    )(term.T)
    return emb, mask_t.T
